# baseline (speedup 1.0000x reference)
import functools
import math

import jax
import jax.numpy as jnp
from jax.experimental import pallas as pl
from jax.experimental.pallas import tpu as pltpu

H = 768
HEADS = 12
DH = H // HEADS
FFN = 4 * H
FC_PAD = 128
NUM_CLASSES = 4
LN_EPS = 1e-12
_VMEM_LIMIT = 48 * 1024 * 1024


def _gelu(x):
    c = math.sqrt(2.0 / math.pi)
    return 0.5 * x * (1.0 + jnp.tanh(c * (x + 0.044715 * x * x * x)))


def _layernorm(y, g, b):
    mean = jnp.mean(y, axis=-1, keepdims=True)
    yc = y - mean
    var = jnp.mean(yc * yc, axis=-1, keepdims=True)
    return yc * jax.lax.rsqrt(var + LN_EPS) * g + b


def _softmax_rows(s):
    mx = jnp.max(s, axis=-1, keepdims=True)
    p = jnp.exp(s - mx)
    return p * pl.reciprocal(jnp.sum(p, axis=-1, keepdims=True), approx=True)


def _mha(qkv, bias):
    qb = qkv[:, :H].astype(jnp.bfloat16)
    kvb = qkv[:, H:].astype(jnp.bfloat16)
    ctxs = []
    for hh in range(HEADS):
        q = qb[:, hh * DH:(hh + 1) * DH]
        k = kvb[:, hh * DH:(hh + 1) * DH]
        v = kvb[:, H + hh * DH:H + (hh + 1) * DH]
        s = jnp.einsum("qd,kd->qk", q, k,
                       preferred_element_type=jnp.float32) + bias
        p = _softmax_rows(s)
        ctxs.append(jnp.dot(p.astype(jnp.bfloat16), v,
                            preferred_element_type=jnp.float32))
    return jnp.concatenate(ctxs, axis=-1)


def _enc_kernel(ids_ref, am_ref, word_ref, pos_ref, typ_ref, eg_ref, eb_ref,
                wqkv_ref, bqkv_ref, wo_ref, bo_ref, g1_ref, bt1_ref,
                w1_ref, b1_ref, w2_ref, b2_ref, g2_ref, bt2_ref,
                pw_ref, pb_ref, fw_ref, fb_ref,
                pooled_ref, logits_ref,
                h_s, *, seq_len, nseq, bw):
    l = pl.program_id(0)
    m = h_s.shape[0]
    nblk = m // bw
    scale = 1.0 / math.sqrt(DH)
    qscale = jnp.concatenate(
        [jnp.full((1, H), scale, jnp.float32),
         jnp.ones((1, 2 * H), jnp.float32)], axis=-1)

    @pl.when(l == 0)
    def _():
        v = word_ref.shape[0]
        oh = jnp.where(
            jax.lax.broadcasted_iota(jnp.int32, (v, m), 0) == ids_ref[...],
            1.0, 0.0).astype(jnp.bfloat16)
        table = word_ref[...]
        hi = table.astype(jnp.bfloat16)
        rem = table - hi.astype(jnp.float32)
        mid = rem.astype(jnp.bfloat16)
        lo = (rem - mid.astype(jnp.float32)).astype(jnp.bfloat16)
        w = (jnp.einsum("vm,vh->mh", oh, hi,
                        preferred_element_type=jnp.float32)
             + jnp.einsum("vm,vh->mh", oh, mid,
                          preferred_element_type=jnp.float32)
             + jnp.einsum("vm,vh->mh", oh, lo,
                          preferred_element_type=jnp.float32))
        pt = pos_ref[...] + typ_ref[0:1, :]
        emb = w + pltpu.repeat(pt, m // seq_len, axis=0)
        h_s[...] = _layernorm(emb, eg_ref[...], eb_ref[...])

    @pl.when(l < pl.num_programs(0) - 1)
    def _():
        row_b = jax.lax.broadcasted_iota(jnp.int32, (bw, bw), 0) // seq_len
        col_b = jax.lax.broadcasted_iota(jnp.int32, (bw, bw), 1) // seq_len
        same_seq = row_b == col_b
        for blk in range(nblk):
            r = slice(blk * bw, (blk + 1) * bw)
            x = h_s[r, :]
            qkv = (jnp.dot(x.astype(jnp.bfloat16), wqkv_ref[...],
                           preferred_element_type=jnp.float32)
                   + bqkv_ref[...]) * qscale
            keep = same_seq & (am_ref[:, r] > 0)
            bias = jnp.where(keep, 0.0, -1e9).astype(jnp.float32)
            ctx = _mha(qkv, bias)
            attn = jnp.dot(ctx.astype(jnp.bfloat16), wo_ref[...],
                           preferred_element_type=jnp.float32)
            h1 = _layernorm(attn + bo_ref[...] + x, g1_ref[...], bt1_ref[...])
            ff = jnp.dot(h1.astype(jnp.bfloat16), w1_ref[...],
                         preferred_element_type=jnp.float32) + b1_ref[...]
            ff = _gelu(ff)
            y2 = jnp.dot(ff.astype(jnp.bfloat16), w2_ref[...],
                         preferred_element_type=jnp.float32) + b2_ref[...] + h1
            h_s[r, :] = _layernorm(y2, g2_ref[...], bt2_ref[...])

    @pl.when(l == pl.num_programs(0) - 1)
    def _():
        x = h_s[...]
        kv = (jnp.dot(x.astype(jnp.bfloat16), wqkv_ref[:, H:],
                      preferred_element_type=jnp.float32)
              + bqkv_ref[:, H:]).astype(jnp.bfloat16)

        cls_x = jnp.concatenate(
            [x[i * seq_len:i * seq_len + 1, :] for i in range(nseq)],
            axis=0)
        q = (jnp.dot(cls_x.astype(jnp.bfloat16), wqkv_ref[:, :H],
                     preferred_element_type=jnp.float32)
             + bqkv_ref[:, :H]) * scale

        nr = HEADS * nseq
        qtile = jnp.concatenate([q] * HEADS, axis=0)
        rowh = jax.lax.broadcasted_iota(jnp.int32, (nr, H), 0) // nseq
        colh = jax.lax.broadcasted_iota(jnp.int32, (nr, H), 1) // DH
        qstack = jnp.where(rowh == colh, qtile, 0.0).astype(jnp.bfloat16)

        s = jnp.einsum("qd,kd->qk", qstack, kv[:, :H],
                       preferred_element_type=jnp.float32)
        rowi = jax.lax.broadcasted_iota(jnp.int32, (nr, m), 0) % nseq
        colb = jax.lax.broadcasted_iota(jnp.int32, (nr, m), 1) // seq_len
        keep = (rowi == colb) & (am_ref[...] > 0)
        bias = jnp.where(keep, 0.0, -1e9).astype(jnp.float32)
        p = _softmax_rows(s + bias)
        ctx_all = jnp.dot(p.astype(jnp.bfloat16), kv[:, H:],
                          preferred_element_type=jnp.float32)
        ctx = jnp.concatenate(
            [ctx_all[hh * nseq:(hh + 1) * nseq, hh * DH:(hh + 1) * DH]
             for hh in range(HEADS)], axis=-1)

        attn = jnp.dot(ctx.astype(jnp.bfloat16), wo_ref[...],
                       preferred_element_type=jnp.float32)
        h1 = _layernorm(attn + bo_ref[...] + cls_x, g1_ref[...], bt1_ref[...])
        ff = jnp.dot(h1.astype(jnp.bfloat16), w1_ref[...],
                     preferred_element_type=jnp.float32) + b1_ref[...]
        ff = _gelu(ff)
        y2 = jnp.dot(ff.astype(jnp.bfloat16), w2_ref[...],
                     preferred_element_type=jnp.float32) + b2_ref[...] + h1
        h2 = _layernorm(y2, g2_ref[...], bt2_ref[...])

        pooled = jnp.tanh(jnp.dot(h2.astype(jnp.bfloat16), pw_ref[...],
                                  preferred_element_type=jnp.float32)
                          + pb_ref[...])
        logits = jnp.dot(pooled.astype(jnp.bfloat16), fw_ref[...],
                         preferred_element_type=jnp.float32) + fb_ref[...]
        pooled_ref[...] = pooled
        logits_ref[...] = logits


def kernel(word_emb, pos_emb, type_emb, emb_ln_g, emb_ln_b, pool_w, pool_b,
           fc_w_pad, fc_b_pad, enc_wqkv, enc_bqkv, enc_wo, enc_bo,
           enc_ln1_g, enc_ln1_b, enc_w1, enc_b1, enc_w2, enc_b2,
           enc_ln2_g, enc_ln2_b, input_ids, attention_mask):
    Bq, Sq = input_ids.shape
    M = Bq * Sq
    L = enc_wqkv.shape[0]
    bw = Sq
    while bw < 256 and M % (2 * bw) == 0:
        bw *= 2

    ids = input_ids.reshape(1, M)
    am = attention_mask.reshape(1, M)
    V = word_emb.shape[0]

    def _const(shape):
        return pl.BlockSpec(shape, lambda l, _n=len(shape): (0,) * _n)

    def _layer(shape):
        return pl.BlockSpec((None,) + shape,
                            lambda l, _n=len(shape): (l,) + (0,) * _n)

    kern = functools.partial(_enc_kernel, seq_len=Sq, nseq=Bq, bw=bw)
    pooled, logits_pad = pl.pallas_call(
        kern,
        out_shape=(jax.ShapeDtypeStruct((Bq, H), jnp.float32),
                   jax.ShapeDtypeStruct((Bq, FC_PAD), jnp.float32)),
        grid_spec=pltpu.PrefetchScalarGridSpec(
            num_scalar_prefetch=0,
            grid=(L,),
            in_specs=[
                _const((1, M)),
                _const((1, M)),
                _const((V, H)),
                _const((Sq, H)),
                _const((2, H)),
                _const((1, H)), _const((1, H)),
                _layer((H, 3 * H)), _layer((1, 3 * H)),
                _layer((H, H)), _layer((1, H)),
                _layer((1, H)), _layer((1, H)),
                _layer((H, FFN)), _layer((1, FFN)),
                _layer((FFN, H)), _layer((1, H)),
                _layer((1, H)), _layer((1, H)),
                _const((H, H)), _const((1, H)),
                _const((H, FC_PAD)), _const((1, FC_PAD)),
            ],
            out_specs=[
                pl.BlockSpec((Bq, H), lambda l: (0, 0)),
                pl.BlockSpec((Bq, FC_PAD), lambda l: (0, 0)),
            ],
            scratch_shapes=[
                pltpu.VMEM((M, H), jnp.float32),
            ],
        ),
        compiler_params=pltpu.CompilerParams(
            dimension_semantics=("arbitrary",),
            vmem_limit_bytes=_VMEM_LIMIT),
    )(ids, am, word_emb, pos_emb, type_emb,
      emb_ln_g.reshape(1, H), emb_ln_b.reshape(1, H),
      enc_wqkv, enc_bqkv, enc_wo, enc_bo, enc_ln1_g, enc_ln1_b,
      enc_w1, enc_b1, enc_w2, enc_b2, enc_ln2_g, enc_ln2_b,
      pool_w, pool_b.reshape(1, H), fc_w_pad, fc_b_pad.reshape(1, FC_PAD))

    logits = logits_pad[:, :NUM_CLASSES]
    return logits, pooled


# speedup vs baseline: 1.3630x; 1.3630x over previous
import functools
import math

import jax
import jax.numpy as jnp
from jax.experimental import pallas as pl
from jax.experimental.pallas import tpu as pltpu

H = 768
HEADS = 12
DH = H // HEADS
FFN = 4 * H
FC_PAD = 128
NUM_CLASSES = 4
LN_EPS = 1e-12
_VMEM_LIMIT = 48 * 1024 * 1024


def _gelu(x):
    c = math.sqrt(2.0 / math.pi)
    return 0.5 * x * (1.0 + jnp.tanh(c * (x + 0.044715 * x * x * x)))


def _layernorm(y, g, b):
    mean = jnp.mean(y, axis=-1, keepdims=True)
    yc = y - mean
    var = jnp.mean(yc * yc, axis=-1, keepdims=True)
    return yc * jax.lax.rsqrt(var + LN_EPS) * g + b


def _softmax_rows(s):
    mx = jnp.max(s, axis=-1, keepdims=True)
    p = jnp.exp(s - mx)
    return p * pl.reciprocal(jnp.sum(p, axis=-1, keepdims=True), approx=True)


def _mha(qkv, bias):
    qb = qkv[:, :H].astype(jnp.bfloat16)
    kvb = qkv[:, H:].astype(jnp.bfloat16)
    ctxs = []
    for hh in range(HEADS):
        q = qb[:, hh * DH:(hh + 1) * DH]
        k = kvb[:, hh * DH:(hh + 1) * DH]
        v = kvb[:, H + hh * DH:H + (hh + 1) * DH]
        s = jnp.einsum("qd,kd->qk", q, k,
                       preferred_element_type=jnp.float32) + bias
        p = _softmax_rows(s)
        ctxs.append(jnp.dot(p.astype(jnp.bfloat16), v,
                            preferred_element_type=jnp.float32))
    return jnp.concatenate(ctxs, axis=-1)


def _enc_kernel(emb_ref, pos_ref, typ_ref, am_ref, eg_ref, eb_ref,
                wqkv_ref, bqkv_ref, wo_ref, bo_ref, g1_ref, bt1_ref,
                w1_ref, b1_ref, w2_ref, b2_ref, g2_ref, bt2_ref,
                pw_ref, pb_ref, fw_ref, fb_ref,
                pooled_ref, logits_ref,
                h_s, *, seq_len, nseq, bw):
    l = pl.program_id(0)
    m = h_s.shape[0]
    nblk = m // bw
    scale = 1.0 / math.sqrt(DH)
    qscale = jnp.concatenate(
        [jnp.full((1, H), scale, jnp.float32),
         jnp.ones((1, 2 * H), jnp.float32)], axis=-1)

    @pl.when(l == 0)
    def _():
        pos = pltpu.repeat(pos_ref[...], m // seq_len, axis=0)
        emb = (emb_ref[...] + pos) + typ_ref[0:1, :]
        h_s[...] = _layernorm(emb, eg_ref[...], eb_ref[...])

    @pl.when(l < pl.num_programs(0) - 1)
    def _():
        row_b = jax.lax.broadcasted_iota(jnp.int32, (bw, bw), 0) // seq_len
        col_b = jax.lax.broadcasted_iota(jnp.int32, (bw, bw), 1) // seq_len
        same_seq = row_b == col_b
        for blk in range(nblk):
            r = slice(blk * bw, (blk + 1) * bw)
            x = h_s[r, :]
            qkv = (jnp.dot(x.astype(jnp.bfloat16), wqkv_ref[...],
                           preferred_element_type=jnp.float32)
                   + bqkv_ref[...]) * qscale
            keep = same_seq & (am_ref[:, r] > 0)
            bias = jnp.where(keep, 0.0, -1e9).astype(jnp.float32)
            ctx = _mha(qkv, bias)
            attn = jnp.dot(ctx.astype(jnp.bfloat16), wo_ref[...],
                           preferred_element_type=jnp.float32)
            h1 = _layernorm(attn + bo_ref[...] + x, g1_ref[...], bt1_ref[...])
            ff = jnp.dot(h1.astype(jnp.bfloat16), w1_ref[...],
                         preferred_element_type=jnp.float32) + b1_ref[...]
            ff = _gelu(ff)
            y2 = jnp.dot(ff.astype(jnp.bfloat16), w2_ref[...],
                         preferred_element_type=jnp.float32) + b2_ref[...] + h1
            h_s[r, :] = _layernorm(y2, g2_ref[...], bt2_ref[...])

    @pl.when(l == pl.num_programs(0) - 1)
    def _():
        x = h_s[...]
        kv = (jnp.dot(x.astype(jnp.bfloat16), wqkv_ref[:, H:],
                      preferred_element_type=jnp.float32)
              + bqkv_ref[:, H:]).astype(jnp.bfloat16)

        cls_x = jnp.concatenate(
            [x[i * seq_len:i * seq_len + 1, :] for i in range(nseq)],
            axis=0)
        q = (jnp.dot(cls_x.astype(jnp.bfloat16), wqkv_ref[:, :H],
                     preferred_element_type=jnp.float32)
             + bqkv_ref[:, :H]) * scale

        nr = HEADS * nseq
        qtile = jnp.concatenate([q] * HEADS, axis=0)
        rowh = jax.lax.broadcasted_iota(jnp.int32, (nr, H), 0) // nseq
        colh = jax.lax.broadcasted_iota(jnp.int32, (nr, H), 1) // DH
        qstack = jnp.where(rowh == colh, qtile, 0.0).astype(jnp.bfloat16)

        s = jnp.einsum("qd,kd->qk", qstack, kv[:, :H],
                       preferred_element_type=jnp.float32)
        rowi = jax.lax.broadcasted_iota(jnp.int32, (nr, m), 0) % nseq
        colb = jax.lax.broadcasted_iota(jnp.int32, (nr, m), 1) // seq_len
        keep = (rowi == colb) & (am_ref[...] > 0)
        bias = jnp.where(keep, 0.0, -1e9).astype(jnp.float32)
        p = _softmax_rows(s + bias)
        ctx_all = jnp.dot(p.astype(jnp.bfloat16), kv[:, H:],
                          preferred_element_type=jnp.float32)
        ctx = jnp.concatenate(
            [ctx_all[hh * nseq:(hh + 1) * nseq, hh * DH:(hh + 1) * DH]
             for hh in range(HEADS)], axis=-1)

        attn = jnp.dot(ctx.astype(jnp.bfloat16), wo_ref[...],
                       preferred_element_type=jnp.float32)
        h1 = _layernorm(attn + bo_ref[...] + cls_x, g1_ref[...], bt1_ref[...])
        ff = jnp.dot(h1.astype(jnp.bfloat16), w1_ref[...],
                     preferred_element_type=jnp.float32) + b1_ref[...]
        ff = _gelu(ff)
        y2 = jnp.dot(ff.astype(jnp.bfloat16), w2_ref[...],
                     preferred_element_type=jnp.float32) + b2_ref[...] + h1
        h2 = _layernorm(y2, g2_ref[...], bt2_ref[...])

        pooled = jnp.tanh(jnp.dot(h2.astype(jnp.bfloat16), pw_ref[...],
                                  preferred_element_type=jnp.float32)
                          + pb_ref[...])
        logits = jnp.dot(pooled.astype(jnp.bfloat16), fw_ref[...],
                         preferred_element_type=jnp.float32) + fb_ref[...]
        pooled_ref[...] = pooled
        logits_ref[...] = logits


def kernel(word_emb, pos_emb, type_emb, emb_ln_g, emb_ln_b, pool_w, pool_b,
           fc_w_pad, fc_b_pad, enc_wqkv, enc_bqkv, enc_wo, enc_bo,
           enc_ln1_g, enc_ln1_b, enc_w1, enc_b1, enc_w2, enc_b2,
           enc_ln2_g, enc_ln2_b, input_ids, attention_mask):
    Bq, Sq = input_ids.shape
    M = Bq * Sq
    L = enc_wqkv.shape[0]
    bw = Sq
    while bw < 256 and M % (2 * bw) == 0:
        bw *= 2

    emb_w = word_emb[input_ids].reshape(M, H)
    am = attention_mask.reshape(1, M)

    def _const(shape):
        return pl.BlockSpec(shape, lambda l, _n=len(shape): (0,) * _n)

    def _layer(shape):
        return pl.BlockSpec((None,) + shape,
                            lambda l, _n=len(shape): (l,) + (0,) * _n)

    kern = functools.partial(_enc_kernel, seq_len=Sq, nseq=Bq, bw=bw)
    pooled, logits_pad = pl.pallas_call(
        kern,
        out_shape=(jax.ShapeDtypeStruct((Bq, H), jnp.float32),
                   jax.ShapeDtypeStruct((Bq, FC_PAD), jnp.float32)),
        grid_spec=pltpu.PrefetchScalarGridSpec(
            num_scalar_prefetch=0,
            grid=(L,),
            in_specs=[
                _const((M, H)),
                _const((Sq, H)),
                _const((2, H)),
                _const((1, M)),
                _const((1, H)), _const((1, H)),
                _layer((H, 3 * H)), _layer((1, 3 * H)),
                _layer((H, H)), _layer((1, H)),
                _layer((1, H)), _layer((1, H)),
                _layer((H, FFN)), _layer((1, FFN)),
                _layer((FFN, H)), _layer((1, H)),
                _layer((1, H)), _layer((1, H)),
                _const((H, H)), _const((1, H)),
                _const((H, FC_PAD)), _const((1, FC_PAD)),
            ],
            out_specs=[
                pl.BlockSpec((Bq, H), lambda l: (0, 0)),
                pl.BlockSpec((Bq, FC_PAD), lambda l: (0, 0)),
            ],
            scratch_shapes=[
                pltpu.VMEM((M, H), jnp.float32),
            ],
        ),
        compiler_params=pltpu.CompilerParams(
            dimension_semantics=("arbitrary",),
            vmem_limit_bytes=_VMEM_LIMIT),
    )(emb_w, pos_emb, type_emb, am,
      emb_ln_g.reshape(1, H), emb_ln_b.reshape(1, H),
      enc_wqkv, enc_bqkv, enc_wo, enc_bo, enc_ln1_g, enc_ln1_b,
      enc_w1, enc_b1, enc_w2, enc_b2, enc_ln2_g, enc_ln2_b,
      pool_w, pool_b.reshape(1, H), fc_w_pad, fc_b_pad.reshape(1, FC_PAD))

    logits = logits_pad[:, :NUM_CLASSES]
    return logits, pooled


# speedup vs baseline: 1.5318x; 1.1239x over previous
import functools
import math

import jax
import jax.numpy as jnp
from jax.experimental import pallas as pl
from jax.experimental.pallas import tpu as pltpu

H = 768
HEADS = 12
DH = H // HEADS
FFN = 4 * H
FC_PAD = 128
NUM_CLASSES = 4
LN_EPS = 1e-12
_VMEM_LIMIT = 48 * 1024 * 1024


def _gelu(x):
    c = math.sqrt(2.0 / math.pi)
    return 0.5 * x * (1.0 + jnp.tanh(c * (x + 0.044715 * x * x * x)))


def _layernorm(y, g, b):
    mean = jnp.mean(y, axis=-1, keepdims=True)
    yc = y - mean
    var = jnp.mean(yc * yc, axis=-1, keepdims=True)
    return yc * jax.lax.rsqrt(var + LN_EPS) * g + b


def _softmax_rows(s):
    mx = jnp.max(s, axis=-1, keepdims=True)
    p = jnp.exp(s - mx)
    return p * pl.reciprocal(jnp.sum(p, axis=-1, keepdims=True), approx=True)


def _mha(qkv, bias):
    qb = qkv[:, :H].astype(jnp.bfloat16)
    kvb = qkv[:, H:].astype(jnp.bfloat16)
    ctxs = []
    for hh in range(HEADS):
        q = qb[:, hh * DH:(hh + 1) * DH]
        k = kvb[:, hh * DH:(hh + 1) * DH]
        v = kvb[:, H + hh * DH:H + (hh + 1) * DH]
        s = jnp.einsum("qd,kd->qk", q, k,
                       preferred_element_type=jnp.float32) + bias
        p = _softmax_rows(s)
        ctxs.append(jnp.dot(p.astype(jnp.bfloat16), v,
                            preferred_element_type=jnp.float32))
    return jnp.concatenate(ctxs, axis=-1)


def _enc_kernel(emb_ref, am_ref, eg_ref, eb_ref,
                wqkv_ref, bqkv_ref, wo_ref, bo_ref, g1_ref, bt1_ref,
                w1_ref, b1_ref, w2_ref, b2_ref, g2_ref, bt2_ref,
                pw_ref, pb_ref, fw_ref, fb_ref,
                pooled_ref, logits_ref,
                h_s, w1_s, w2_s, sem1, sem2, *, seq_len, nseq, bw):
    l = pl.program_id(0)
    m = h_s.shape[0]
    nl = w1_s.shape[0]
    nblk = m // bw
    scale = 1.0 / math.sqrt(DH)
    qscale = jnp.concatenate(
        [jnp.full((1, H), scale, jnp.float32),
         jnp.ones((1, 2 * H), jnp.float32)], axis=-1)

    @pl.when(l == 0)
    def _():
        for ll in range(nl):
            pltpu.make_async_copy(w1_ref.at[ll], w1_s.at[ll],
                                  sem1.at[ll]).start()
            pltpu.make_async_copy(w2_ref.at[ll], w2_s.at[ll],
                                  sem2.at[ll]).start()
        h_s[...] = _layernorm(emb_ref[...], eg_ref[...], eb_ref[...])

    @pl.when(l < pl.num_programs(0) - 1)
    def _():
        row_b = jax.lax.broadcasted_iota(jnp.int32, (bw, bw), 0) // seq_len
        col_b = jax.lax.broadcasted_iota(jnp.int32, (bw, bw), 1) // seq_len
        same_seq = row_b == col_b
        for blk in range(nblk):
            r = slice(blk * bw, (blk + 1) * bw)
            x = h_s[r, :]
            qkv = (jnp.dot(x.astype(jnp.bfloat16), wqkv_ref[...],
                           preferred_element_type=jnp.float32)
                   + bqkv_ref[...]) * qscale
            keep = same_seq & (am_ref[:, r] > 0.5)
            bias = jnp.where(keep, 0.0, -1e9).astype(jnp.float32)
            ctx = _mha(qkv, bias)
            attn = jnp.dot(ctx.astype(jnp.bfloat16), wo_ref[...],
                           preferred_element_type=jnp.float32)
            h1 = _layernorm(attn + bo_ref[...] + x, g1_ref[...], bt1_ref[...])
            if blk == 0:
                pltpu.make_async_copy(w1_s.at[l], w1_s.at[l],
                                      sem1.at[l]).wait()
                pltpu.make_async_copy(w2_s.at[l], w2_s.at[l],
                                      sem2.at[l]).wait()
            ff = jnp.dot(h1.astype(jnp.bfloat16), w1_s[l],
                         preferred_element_type=jnp.float32) + b1_ref[...]
            ff = _gelu(ff)
            y2 = jnp.dot(ff.astype(jnp.bfloat16), w2_s[l],
                         preferred_element_type=jnp.float32) + b2_ref[...] + h1
            h_s[r, :] = _layernorm(y2, g2_ref[...], bt2_ref[...])

    @pl.when(l == pl.num_programs(0) - 1)
    def _():
        x = h_s[...]
        kv = (jnp.dot(x.astype(jnp.bfloat16), wqkv_ref[:, H:],
                      preferred_element_type=jnp.float32)
              + bqkv_ref[:, H:]).astype(jnp.bfloat16)

        cls_x = jnp.concatenate(
            [x[i * seq_len:i * seq_len + 1, :] for i in range(nseq)],
            axis=0)
        q = (jnp.dot(cls_x.astype(jnp.bfloat16), wqkv_ref[:, :H],
                     preferred_element_type=jnp.float32)
             + bqkv_ref[:, :H]) * scale

        nr = HEADS * nseq
        qtile = jnp.concatenate([q] * HEADS, axis=0)
        rowh = jax.lax.broadcasted_iota(jnp.int32, (nr, H), 0) // nseq
        colh = jax.lax.broadcasted_iota(jnp.int32, (nr, H), 1) // DH
        qstack = jnp.where(rowh == colh, qtile, 0.0).astype(jnp.bfloat16)

        s = jnp.einsum("qd,kd->qk", qstack, kv[:, :H],
                       preferred_element_type=jnp.float32)
        rowi = jax.lax.broadcasted_iota(jnp.int32, (nr, m), 0) % nseq
        colb = jax.lax.broadcasted_iota(jnp.int32, (nr, m), 1) // seq_len
        keep = (rowi == colb) & (am_ref[...] > 0.5)
        bias = jnp.where(keep, 0.0, -1e9).astype(jnp.float32)
        p = _softmax_rows(s + bias)
        ctx_all = jnp.dot(p.astype(jnp.bfloat16), kv[:, H:],
                          preferred_element_type=jnp.float32)
        ctx = jnp.concatenate(
            [ctx_all[hh * nseq:(hh + 1) * nseq, hh * DH:(hh + 1) * DH]
             for hh in range(HEADS)], axis=-1)

        attn = jnp.dot(ctx.astype(jnp.bfloat16), wo_ref[...],
                       preferred_element_type=jnp.float32)
        h1 = _layernorm(attn + bo_ref[...] + cls_x, g1_ref[...], bt1_ref[...])
        pltpu.make_async_copy(w1_s.at[l], w1_s.at[l], sem1.at[l]).wait()
        pltpu.make_async_copy(w2_s.at[l], w2_s.at[l], sem2.at[l]).wait()
        ff = jnp.dot(h1.astype(jnp.bfloat16), w1_s[l],
                     preferred_element_type=jnp.float32) + b1_ref[...]
        ff = _gelu(ff)
        y2 = jnp.dot(ff.astype(jnp.bfloat16), w2_s[l],
                     preferred_element_type=jnp.float32) + b2_ref[...] + h1
        h2 = _layernorm(y2, g2_ref[...], bt2_ref[...])

        pooled = jnp.tanh(jnp.dot(h2.astype(jnp.bfloat16), pw_ref[...],
                                  preferred_element_type=jnp.float32)
                          + pb_ref[...])
        logits = jnp.dot(pooled.astype(jnp.bfloat16), fw_ref[...],
                         preferred_element_type=jnp.float32) + fb_ref[...]
        pooled_ref[...] = pooled
        logits_ref[...] = logits


def kernel(word_emb, pos_emb, type_emb, emb_ln_g, emb_ln_b, pool_w, pool_b,
           fc_w_pad, fc_b_pad, enc_wqkv, enc_bqkv, enc_wo, enc_bo,
           enc_ln1_g, enc_ln1_b, enc_w1, enc_b1, enc_w2, enc_b2,
           enc_ln2_g, enc_ln2_b, input_ids, attention_mask):
    Bq, Sq = input_ids.shape
    M = Bq * Sq
    L = enc_wqkv.shape[0]
    bw = Sq
    while bw < 256 and M % (2 * bw) == 0:
        bw *= 2

    emb = (word_emb[input_ids] + pos_emb[:Sq][None, :, :]
           + type_emb[0][None, None, :]).reshape(M, H).astype(jnp.float32)
    am = attention_mask.astype(jnp.float32).reshape(1, M)

    def _const(shape):
        return pl.BlockSpec(shape, lambda l, _n=len(shape): (0,) * _n)

    def _layer(shape):
        return pl.BlockSpec((None,) + shape,
                            lambda l, _n=len(shape): (l,) + (0,) * _n)

    kern = functools.partial(_enc_kernel, seq_len=Sq, nseq=Bq, bw=bw)
    pooled, logits_pad = pl.pallas_call(
        kern,
        out_shape=(jax.ShapeDtypeStruct((Bq, H), jnp.float32),
                   jax.ShapeDtypeStruct((Bq, FC_PAD), jnp.float32)),
        grid_spec=pltpu.PrefetchScalarGridSpec(
            num_scalar_prefetch=0,
            grid=(L,),
            in_specs=[
                _const((M, H)),
                _const((1, M)),
                _const((1, H)), _const((1, H)),
                _layer((H, 3 * H)), _layer((1, 3 * H)),
                _layer((H, H)), _layer((1, H)),
                _layer((1, H)), _layer((1, H)),
                pl.BlockSpec(memory_space=pl.ANY),
                _layer((1, FFN)),
                pl.BlockSpec(memory_space=pl.ANY),
                _layer((1, H)),
                _layer((1, H)), _layer((1, H)),
                _const((H, H)), _const((1, H)),
                _const((H, FC_PAD)), _const((1, FC_PAD)),
            ],
            out_specs=[
                pl.BlockSpec((Bq, H), lambda l: (0, 0)),
                pl.BlockSpec((Bq, FC_PAD), lambda l: (0, 0)),
            ],
            scratch_shapes=[
                pltpu.VMEM((M, H), jnp.float32),
                pltpu.VMEM((L, H, FFN), jnp.bfloat16),
                pltpu.VMEM((L, FFN, H), jnp.bfloat16),
                pltpu.SemaphoreType.DMA((L,)),
                pltpu.SemaphoreType.DMA((L,)),
            ],
        ),
        compiler_params=pltpu.CompilerParams(
            dimension_semantics=("arbitrary",),
            vmem_limit_bytes=_VMEM_LIMIT),
    )(emb, am, emb_ln_g.reshape(1, H), emb_ln_b.reshape(1, H),
      enc_wqkv, enc_bqkv, enc_wo, enc_bo, enc_ln1_g, enc_ln1_b,
      enc_w1, enc_b1, enc_w2, enc_b2, enc_ln2_g, enc_ln2_b,
      pool_w, pool_b.reshape(1, H), fc_w_pad, fc_b_pad.reshape(1, FC_PAD))

    logits = logits_pad[:, :NUM_CLASSES]
    return logits, pooled
